# Initial kernel scaffold; baseline (speedup 1.0000x reference)
import jax
import jax.numpy as jnp
from jax.experimental import pallas as pl


def kernel(input, emb_table):
    raise NotImplementedError("write your pallas kernel here")


# speedup vs baseline: 26.4018x; 26.4018x over previous
import functools

import jax
import jax.numpy as jnp
from jax import lax
from jax.experimental import pallas as pl
from jax.experimental.pallas import tpu as pltpu
from jax.experimental.pallas import tpu_sc as plsc

B = 4096
L = 100
E = 64
D = L * E
LANES = 16
NC = 2
NS = 16
NW = NC * NS
ROWS_PER_W = B // NW
RBLK = 8
NBLK = ROWS_PER_W // RBLK


@functools.partial(
    pl.kernel,
    mesh=plsc.VectorSubcoreMesh(core_axis_name="c", subcore_axis_name="s"),
    out_type=jax.ShapeDtypeStruct((B * D,), jnp.float32),
    compiler_params=pltpu.CompilerParams(needs_layout_passes=False),
    scratch_types=[
        pltpu.VMEM((ROWS_PER_W, L), jnp.float32),
        pltpu.VMEM((D,), jnp.float32),
        pltpu.VMEM((RBLK * D,), jnp.float32),
        pltpu.VMEM((RBLK * D,), jnp.float32),
        pltpu.SemaphoreType.DMA,
        pltpu.SemaphoreType.DMA,
        pltpu.SemaphoreType.DMA,
    ],
)
def _sc_expand(in_hbm, tab_hbm, out_hbm, in_v, tab_v, out_a, out_b,
               sem_a, sem_b, sem_in):
    wid = lax.axis_index("s") * NC + lax.axis_index("c")
    row0 = wid * ROWS_PER_W
    tab_cp = pltpu.async_copy(tab_hbm, tab_v, sem_a)
    in_cp = pltpu.async_copy(in_hbm.at[pl.ds(row0, ROWS_PER_W)], in_v, sem_in)
    tab_cp.wait()
    in_cp.wait()
    bufs = (out_a, out_b)
    sems = (sem_a, sem_b)
    copies = [None, None]
    for blk in range(NBLK):
        slot = blk % 2
        buf = bufs[slot]
        if copies[slot] is not None:
            copies[slot].wait()

        @plsc.parallel_loop(0, L, 1, unroll=2)
        def ibody(i, blk=blk, buf=buf):
            o = i * E
            ts = [tab_v[pl.ds(o + c * LANES, LANES)] for c in range(E // LANES)]
            iv = jnp.full((LANES,), 0, jnp.int32) + i
            for r in range(RBLK):
                rv = jnp.full((LANES,), blk * RBLK + r, jnp.int32)
                s = plsc.load_gather(in_v, [rv, iv])
                for c in range(E // LANES):
                    buf[pl.ds(r * D + o + c * LANES, LANES)] = ts[c] * s
        copies[slot] = pltpu.async_copy(
            buf, out_hbm.at[pl.ds((row0 + blk * RBLK) * D, RBLK * D)], sems[slot]
        )
    copies[0].wait()
    copies[1].wait()


def kernel(input, emb_table):
    flat = _sc_expand(input, emb_table.reshape(-1))
    return flat.reshape(B, 1, D)
